# baseline (speedup 1.0000x reference)
import functools

import jax
import jax.numpy as jnp
import numpy as np
from jax import lax
from jax.experimental import pallas as pl
from jax.experimental.pallas import tpu as pltpu

_B = 64
_N = 1_000_000
_C = 65536
_NBLK = (_N + _C - 1) // _C
_LANES = 128
_CHUNKS = _C // _LANES

_KS0 = np.uint32(0)
_KS1 = np.uint32(42)
_KS2 = np.uint32(0 ^ 42 ^ 0x1BD11BDA)
_ROTS = ((13, 15, 26, 6), (17, 29, 16, 24))

_TINY = np.float32(np.finfo(np.float32).tiny)
_NEG_LN2 = np.float32(-0.6931471805599453)
_NEG_INF = np.float32(-np.inf)
_BIG_I32 = np.int32(2**31 - 1)


def _rotl(v, r):
    return lax.shift_left(v, np.uint32(r)) ^ lax.shift_right_logical(
        v, np.uint32(32 - r)
    )


def _threefry_uniform(x1):
    x0 = x1
    x1 = _rotl(x1, _ROTS[0][0]) ^ x0
    first = True
    ks = (_KS0, _KS1, _KS2)
    for i in range(5):
        for r in _ROTS[i % 2]:
            if first:
                first = False
                continue
            x0 = x0 + x1
            x1 = _rotl(x1, r) ^ x0
        kx = ks[(i + 1) % 3]
        if kx:
            x0 = x0 + kx
        x1 = x1 + np.uint32(ks[(i + 2) % 3] + np.uint32(i + 1))
    bits = x0 ^ x1

    float_bits = lax.shift_right_logical(bits, np.uint32(9)) | np.uint32(0x3F800000)
    floats = lax.bitcast_convert_type(float_bits, jnp.float32) - np.float32(1.0)
    return lax.max(floats, _TINY)


def _sample_block_kernel(logits_ref, out_ref, bv_ref, bc_ref):
    j = pl.program_id(0)

    valid = jnp.minimum(_C, _N - j * _C)

    row = lax.broadcasted_iota(jnp.int32, (_B, _LANES), 0)
    lane = lax.broadcasted_iota(jnp.int32, (_B, _LANES), 1)
    x1_base = (row * _N + lane).astype(jnp.uint32) + (
        (j * _C).astype(jnp.uint32) + _KS1
    )

    @pl.when(j == 0)
    def _init():
        bv_ref[...] = jnp.full((_B, _LANES), _NEG_INF, jnp.float32)
        bc_ref[...] = jnp.zeros((_B, _LANES), jnp.int32)
        out_ref[...] = jnp.zeros((_B, 1), jnp.int32)

    def score(u, off, bv, bc, masked):
        g = jnp.log2(jnp.log2(u) * _NEG_LN2) * _NEG_LN2
        s = g + logits_ref[:, pl.ds(off, _LANES)]
        if masked:
            s = jnp.where(lane < valid - off, s, _NEG_INF)
        upd = s > bv
        bv = jnp.maximum(s, bv)
        bc = jnp.where(upd, j * _C + off, bc)
        return bv, bc

    def run(masked):
        def body(k, carry):
            bv, bc, u_prev = carry
            off = k * _LANES
            u = _threefry_uniform(x1_base + off.astype(jnp.uint32))
            bv, bc = score(u_prev, off - _LANES, bv, bc, masked)
            return bv, bc, u

        u0 = _threefry_uniform(x1_base)
        bv, bc, u_last = lax.fori_loop(
            1, _CHUNKS, body, (bv_ref[...], bc_ref[...], u0), unroll=8
        )
        bv, bc = score(u_last, (_CHUNKS - 1) * _LANES, bv, bc, masked)
        bv_ref[...] = bv
        bc_ref[...] = bc

    @pl.when(j < _NBLK - 1)
    def _full():
        run(masked=False)

    @pl.when(j == _NBLK - 1)
    def _tail():
        run(masked=True)

        col = bc_ref[...] + lane
        bv = bv_ref[...]
        m = jnp.max(bv, axis=1, keepdims=True)
        idx = jnp.min(jnp.where(bv == m, col, _BIG_I32), axis=1, keepdims=True)
        out_ref[...] = idx


@functools.partial(jax.jit, static_argnames=("interpret",))
def _sample(logits, interpret=False):
    out = pl.pallas_call(
        _sample_block_kernel,
        grid=(_NBLK,),
        in_specs=[pl.BlockSpec((_B, _C), lambda j: (0, j))],
        out_specs=pl.BlockSpec((_B, 1), lambda j: (0, 0)),
        out_shape=jax.ShapeDtypeStruct((_B, 1), jnp.int32),
        scratch_shapes=[
            pltpu.VMEM((_B, _LANES), jnp.float32),
            pltpu.VMEM((_B, _LANES), jnp.int32),
        ],
        interpret=interpret,
    )(logits)
    return out.reshape(_B).astype(jnp.int64)


def kernel(logits):
    return _sample(logits)


# speedup vs baseline: 1.0338x; 1.0338x over previous
import functools

import jax
import jax.numpy as jnp
import numpy as np
from jax import lax
from jax.experimental import pallas as pl
from jax.experimental.pallas import tpu as pltpu

_B = 64
_N = 1_000_000
_C = 32768
_NBLK = (_N + _C - 1) // _C
_LANES = 128
_CHUNKS = _C // _LANES

_KS0 = np.uint32(0)
_KS1 = np.uint32(42)
_KS2 = np.uint32(0 ^ 42 ^ 0x1BD11BDA)
_ROTS = ((13, 15, 26, 6), (17, 29, 16, 24))

_TINY = np.float32(np.finfo(np.float32).tiny)
_NEG_LN2 = np.float32(-0.6931471805599453)
_NEG_INF = np.float32(-np.inf)
_BIG_I32 = np.int32(2**31 - 1)


def _rotl(v, r):
    return lax.shift_left(v, np.uint32(r)) ^ lax.shift_right_logical(
        v, np.uint32(32 - r)
    )


def _threefry_uniform(x1):
    x0 = x1
    x1 = _rotl(x1, _ROTS[0][0]) ^ x0
    first = True
    ks = (_KS0, _KS1, _KS2)
    for i in range(5):
        for r in _ROTS[i % 2]:
            if first:
                first = False
                continue
            x0 = x0 + x1
            x1 = _rotl(x1, r) ^ x0
        kx = ks[(i + 1) % 3]
        if kx:
            x0 = x0 + kx
        x1 = x1 + np.uint32(ks[(i + 2) % 3] + np.uint32(i + 1))
    bits = x0 ^ x1

    float_bits = lax.shift_right_logical(bits, np.uint32(9)) | np.uint32(0x3F800000)
    floats = lax.bitcast_convert_type(float_bits, jnp.float32) - np.float32(1.0)
    return lax.max(floats, _TINY)


def _sample_block_kernel(logits_ref, out_ref, bv_ref, bc_ref):
    j = pl.program_id(0)

    valid = jnp.minimum(_C, _N - j * _C)

    row = lax.broadcasted_iota(jnp.int32, (_B, _LANES), 0)
    lane = lax.broadcasted_iota(jnp.int32, (_B, _LANES), 1)
    x1_base = (row * _N + lane).astype(jnp.uint32) + (
        (j * _C).astype(jnp.uint32) + _KS1
    )

    @pl.when(j == 0)
    def _init():
        bv_ref[...] = jnp.full((_B, _LANES), _NEG_INF, jnp.float32)
        bc_ref[...] = jnp.zeros((_B, _LANES), jnp.int32)
        out_ref[...] = jnp.zeros((_B, 1), jnp.int32)

    def score(u, off, bv, bc, masked):
        g = jnp.log2(jnp.log2(u) * _NEG_LN2) * _NEG_LN2
        s = g + logits_ref[:, pl.ds(off, _LANES)]
        if masked:
            s = jnp.where(lane < valid - off, s, _NEG_INF)
        upd = s > bv
        bv = jnp.maximum(s, bv)
        bc = jnp.where(upd, j * _C + off, bc)
        return bv, bc

    def run(masked):
        def body(k, carry):
            bv, bc, u_prev = carry
            off = k * _LANES
            u = _threefry_uniform(x1_base + off.astype(jnp.uint32))
            bv, bc = score(u_prev, off - _LANES, bv, bc, masked)
            return bv, bc, u

        u0 = _threefry_uniform(x1_base)
        bv, bc, u_last = lax.fori_loop(
            1, _CHUNKS, body, (bv_ref[...], bc_ref[...], u0), unroll=8
        )
        bv, bc = score(u_last, (_CHUNKS - 1) * _LANES, bv, bc, masked)
        bv_ref[...] = bv
        bc_ref[...] = bc

    @pl.when(j < _NBLK - 1)
    def _full():
        run(masked=False)

    @pl.when(j == _NBLK - 1)
    def _tail():
        run(masked=True)

        col = bc_ref[...] + lane
        bv = bv_ref[...]
        m = jnp.max(bv, axis=1, keepdims=True)
        idx = jnp.min(jnp.where(bv == m, col, _BIG_I32), axis=1, keepdims=True)
        out_ref[...] = idx


@functools.partial(jax.jit, static_argnames=("interpret",))
def _sample(logits, interpret=False):
    out = pl.pallas_call(
        _sample_block_kernel,
        grid=(_NBLK,),
        in_specs=[pl.BlockSpec((_B, _C), lambda j: (0, j))],
        out_specs=pl.BlockSpec((_B, 1), lambda j: (0, 0)),
        out_shape=jax.ShapeDtypeStruct((_B, 1), jnp.int32),
        scratch_shapes=[
            pltpu.VMEM((_B, _LANES), jnp.float32),
            pltpu.VMEM((_B, _LANES), jnp.int32),
        ],
        interpret=interpret,
    )(logits)
    return out.reshape(_B).astype(jnp.int64)


def kernel(logits):
    return _sample(logits)


# speedup vs baseline: 1.0522x; 1.0178x over previous
import functools

import jax
import jax.numpy as jnp
import numpy as np
from jax import lax
from jax.experimental import pallas as pl
from jax.experimental.pallas import tpu as pltpu

_B = 64
_N = 1_000_000
_C = 32768
_NBLK = (_N + _C - 1) // _C
_LANES = 128
_CHUNKS = _C // _LANES

_KS0 = np.uint32(0)
_KS1 = np.uint32(42)
_KS2 = np.uint32(0 ^ 42 ^ 0x1BD11BDA)
_ROTS = ((13, 15, 26, 6), (17, 29, 16, 24))

_TINY = np.float32(np.finfo(np.float32).tiny)
_NEG_LN2 = np.float32(-0.6931471805599453)
_NEG_INF = np.float32(-np.inf)
_BIG_I32 = np.int32(2**31 - 1)


def _rotl(v, r):
    return lax.shift_left(v, np.uint32(r)) ^ lax.shift_right_logical(
        v, np.uint32(32 - r)
    )


def _threefry_uniform(x1):
    x0 = x1
    x1 = _rotl(x1, _ROTS[0][0]) ^ x0
    first = True
    ks = (_KS0, _KS1, _KS2)
    for i in range(5):
        for r in _ROTS[i % 2]:
            if first:
                first = False
                continue
            x0 = x0 + x1
            x1 = _rotl(x1, r) ^ x0
        kx = ks[(i + 1) % 3]
        if kx:
            x0 = x0 + kx
        x1 = x1 + np.uint32(ks[(i + 2) % 3] + np.uint32(i + 1))
    bits = x0 ^ x1

    float_bits = lax.shift_right_logical(bits, np.uint32(9)) | np.uint32(0x3F800000)
    floats = lax.bitcast_convert_type(float_bits, jnp.float32) - np.float32(1.0)
    return lax.max(floats, _TINY)


def _sample_block_kernel(logits_ref, out_ref, bv_ref, bc_ref):
    j = pl.program_id(0)

    valid = jnp.minimum(_C, _N - j * _C)

    row = lax.broadcasted_iota(jnp.int32, (_B, _LANES), 0)
    lane = lax.broadcasted_iota(jnp.int32, (_B, _LANES), 1)
    x1_base = (row * _N + lane).astype(jnp.uint32) + (
        (j * _C).astype(jnp.uint32) + _KS1
    )

    @pl.when(j == 0)
    def _init():
        bv_ref[...] = jnp.full((_B, _LANES), _NEG_INF, jnp.float32)
        bc_ref[...] = jnp.zeros((_B, _LANES), jnp.int32)
        out_ref[...] = jnp.zeros((_B, 1), jnp.int32)

    def score(u, off, bv, bc, masked):
        g = -jnp.log(-jnp.log(u))
        s = g + logits_ref[:, pl.ds(off, _LANES)]
        if masked:
            s = jnp.where(lane < valid - off, s, _NEG_INF)
        upd = s > bv
        bv = jnp.maximum(s, bv)
        bc = jnp.where(upd, j * _C + off, bc)
        return bv, bc

    def run(masked):
        def body(k, carry):
            bv, bc, u_prev = carry
            off = k * _LANES
            u = _threefry_uniform(x1_base + off.astype(jnp.uint32))
            bv, bc = score(u_prev, off - _LANES, bv, bc, masked)
            return bv, bc, u

        u0 = _threefry_uniform(x1_base)
        bv, bc, u_last = lax.fori_loop(
            1, _CHUNKS, body, (bv_ref[...], bc_ref[...], u0), unroll=16
        )
        bv, bc = score(u_last, (_CHUNKS - 1) * _LANES, bv, bc, masked)
        bv_ref[...] = bv
        bc_ref[...] = bc

    @pl.when(j < _NBLK - 1)
    def _full():
        run(masked=False)

    @pl.when(j == _NBLK - 1)
    def _tail():
        run(masked=True)

        col = bc_ref[...] + lane
        bv = bv_ref[...]
        m = jnp.max(bv, axis=1, keepdims=True)
        idx = jnp.min(jnp.where(bv == m, col, _BIG_I32), axis=1, keepdims=True)
        out_ref[...] = idx


@functools.partial(jax.jit, static_argnames=("interpret",))
def _sample(logits, interpret=False):
    out = pl.pallas_call(
        _sample_block_kernel,
        grid=(_NBLK,),
        in_specs=[pl.BlockSpec((_B, _C), lambda j: (0, j))],
        out_specs=pl.BlockSpec((_B, 1), lambda j: (0, 0)),
        out_shape=jax.ShapeDtypeStruct((_B, 1), jnp.int32),
        scratch_shapes=[
            pltpu.VMEM((_B, _LANES), jnp.float32),
            pltpu.VMEM((_B, _LANES), jnp.int32),
        ],
        interpret=interpret,
    )(logits)
    return out.reshape(_B).astype(jnp.int64)


def kernel(logits):
    return _sample(logits)


# speedup vs baseline: 1.0654x; 1.0126x over previous
import functools

import jax
import jax.numpy as jnp
import numpy as np
from jax import lax
from jax.experimental import pallas as pl
from jax.experimental.pallas import tpu as pltpu

_B = 64
_N = 1_000_000
_C = 65536
_NBLK = (_N + _C - 1) // _C
_LANES = 128
_CHUNKS = _C // _LANES

_KS0 = np.uint32(0)
_KS1 = np.uint32(42)
_KS2 = np.uint32(0 ^ 42 ^ 0x1BD11BDA)
_ROTS = ((13, 15, 26, 6), (17, 29, 16, 24))

_TINY = np.float32(np.finfo(np.float32).tiny)
_NEG_LN2 = np.float32(-0.6931471805599453)
_NEG_INF = np.float32(-np.inf)
_BIG_I32 = np.int32(2**31 - 1)


def _rotl(v, r):
    return lax.shift_left(v, np.uint32(r)) ^ lax.shift_right_logical(
        v, np.uint32(32 - r)
    )


def _threefry_uniform(x1):
    x0 = x1
    x1 = _rotl(x1, _ROTS[0][0]) ^ x0
    first = True
    ks = (_KS0, _KS1, _KS2)
    for i in range(5):
        for r in _ROTS[i % 2]:
            if first:
                first = False
                continue
            x0 = x0 + x1
            x1 = _rotl(x1, r) ^ x0
        kx = ks[(i + 1) % 3]
        if kx:
            x0 = x0 + kx
        x1 = x1 + np.uint32(ks[(i + 2) % 3] + np.uint32(i + 1))
    bits = x0 ^ x1

    float_bits = lax.shift_right_logical(bits, np.uint32(9)) | np.uint32(0x3F800000)
    floats = lax.bitcast_convert_type(float_bits, jnp.float32) - np.float32(1.0)
    return lax.max(floats, _TINY)


def _sample_block_kernel(logits_ref, out_ref, bv_ref, bc_ref):
    j = pl.program_id(0)

    valid = jnp.minimum(_C, _N - j * _C)

    row = lax.broadcasted_iota(jnp.int32, (_B, _LANES), 0)
    lane = lax.broadcasted_iota(jnp.int32, (_B, _LANES), 1)
    x1_base = (row * _N + lane).astype(jnp.uint32) + (
        (j * _C).astype(jnp.uint32) + _KS1
    )

    @pl.when(j == 0)
    def _init():
        bv_ref[...] = jnp.full((_B, _LANES), _NEG_INF, jnp.float32)
        bc_ref[...] = jnp.zeros((_B, _LANES), jnp.int32)
        out_ref[...] = jnp.zeros((_B, 1), jnp.int32)

    def score(u, off, bv, bc, masked):
        g = -jnp.log(-jnp.log(u))
        s = g + logits_ref[:, pl.ds(off, _LANES)]
        if masked:
            s = jnp.where(lane < valid - off, s, _NEG_INF)
        upd = s > bv
        bv = jnp.maximum(s, bv)
        bc = jnp.where(upd, j * _C + off, bc)
        return bv, bc

    def run(masked):
        def body(k, carry):
            bv, bc, u_prev = carry
            off = k * _LANES
            u = _threefry_uniform(x1_base + off.astype(jnp.uint32))
            bv, bc = score(u_prev, off - _LANES, bv, bc, masked)
            return bv, bc, u

        nch = _CHUNKS if not masked else (valid + _LANES - 1) // _LANES
        u0 = _threefry_uniform(x1_base)
        bv, bc, u_last = lax.fori_loop(
            1,
            nch,
            body,
            (bv_ref[...], bc_ref[...], u0),
            unroll=16 if not masked else None,
        )
        bv, bc = score(u_last, (nch - 1) * _LANES, bv, bc, masked)
        bv_ref[...] = bv
        bc_ref[...] = bc

    @pl.when(j < _NBLK - 1)
    def _full():
        run(masked=False)

    @pl.when(j == _NBLK - 1)
    def _tail():
        run(masked=True)

        col = bc_ref[...] + lane
        bv = bv_ref[...]
        m = jnp.max(bv, axis=1, keepdims=True)
        idx = jnp.min(jnp.where(bv == m, col, _BIG_I32), axis=1, keepdims=True)
        out_ref[...] = idx


@functools.partial(jax.jit, static_argnames=("interpret",))
def _sample(logits, interpret=False):
    out = pl.pallas_call(
        _sample_block_kernel,
        grid=(_NBLK,),
        in_specs=[pl.BlockSpec((_B, _C), lambda j: (0, j))],
        out_specs=pl.BlockSpec((_B, 1), lambda j: (0, 0)),
        out_shape=jax.ShapeDtypeStruct((_B, 1), jnp.int32),
        scratch_shapes=[
            pltpu.VMEM((_B, _LANES), jnp.float32),
            pltpu.VMEM((_B, _LANES), jnp.int32),
        ],
        interpret=interpret,
    )(logits)
    return out.reshape(_B).astype(jnp.int64)


def kernel(logits):
    return _sample(logits)


# speedup vs baseline: 1.0672x; 1.0017x over previous
import functools

import jax
import jax.numpy as jnp
import numpy as np
from jax import lax
from jax.experimental import pallas as pl
from jax.experimental.pallas import tpu as pltpu

_B = 64
_N = 1_000_000
_C = 65536
_NBLK = (_N + _C - 1) // _C
_LANES = 128
_CHUNKS = _C // _LANES

_KS0 = np.uint32(0)
_KS1 = np.uint32(42)
_KS2 = np.uint32(0 ^ 42 ^ 0x1BD11BDA)
_ROTS = ((13, 15, 26, 6), (17, 29, 16, 24))

_TINY = np.float32(np.finfo(np.float32).tiny)
_NEG_LN2 = np.float32(-0.6931471805599453)
_NEG_INF = np.float32(-np.inf)
_BIG_I32 = np.int32(2**31 - 1)


def _rotl(v, r):
    return lax.shift_left(v, np.uint32(r)) ^ lax.shift_right_logical(
        v, np.uint32(32 - r)
    )


def _threefry_uniform(x1):
    x0 = x1
    x1 = _rotl(x1, _ROTS[0][0]) ^ x0
    first = True
    ks = (_KS0, _KS1, _KS2)
    for i in range(5):
        for r in _ROTS[i % 2]:
            if first:
                first = False
                continue
            x0 = x0 + x1
            x1 = _rotl(x1, r) ^ x0
        kx = ks[(i + 1) % 3]
        if kx:
            x0 = x0 + kx
        x1 = x1 + np.uint32(ks[(i + 2) % 3] + np.uint32(i + 1))
    bits = x0 ^ x1

    float_bits = lax.shift_right_logical(bits, np.uint32(9)) | np.uint32(0x3F800000)
    floats = lax.bitcast_convert_type(float_bits, jnp.float32) - np.float32(1.0)
    return lax.max(floats, _TINY)


def _sample_block_kernel(logits_ref, out_ref, bv_ref, bc_ref):
    j = pl.program_id(0)

    valid = jnp.minimum(_C, _N - j * _C)

    row = lax.broadcasted_iota(jnp.int32, (_B, _LANES), 0)
    lane = lax.broadcasted_iota(jnp.int32, (_B, _LANES), 1)
    x1_base = (row * _N + lane).astype(jnp.uint32) + (
        (j * _C).astype(jnp.uint32) + _KS1
    )

    @pl.when(j == 0)
    def _init():
        bv_ref[...] = jnp.full((_B, _LANES), _NEG_INF, jnp.float32)
        bc_ref[...] = jnp.zeros((_B, _LANES), jnp.int32)
        out_ref[...] = jnp.zeros((_B, 1), jnp.int32)

    def score(u, off, bv, bc, masked):
        g = -jnp.log(-jnp.log(u))
        s = g + logits_ref[:, pl.ds(off, _LANES)]
        if masked:
            s = jnp.where(lane < valid - off, s, _NEG_INF)
        upd = s > bv
        bv = jnp.maximum(s, bv)
        bc = jnp.where(upd, j * _C + off, bc)
        return bv, bc

    def run(masked):
        def body(k, carry):
            bv, bc, u_prev = carry
            off = k * _LANES
            u = _threefry_uniform(x1_base + off.astype(jnp.uint32))
            bv, bc = score(u_prev, off - _LANES, bv, bc, masked)
            return bv, bc, u

        nch = _CHUNKS if not masked else (valid + _LANES - 1) // _LANES
        u0 = _threefry_uniform(x1_base)
        bv, bc, u_last = lax.fori_loop(
            1,
            nch,
            body,
            (bv_ref[...], bc_ref[...], u0),
            unroll=32 if not masked else None,
        )
        bv, bc = score(u_last, (nch - 1) * _LANES, bv, bc, masked)
        bv_ref[...] = bv
        bc_ref[...] = bc

    @pl.when(j < _NBLK - 1)
    def _full():
        run(masked=False)

    @pl.when(j == _NBLK - 1)
    def _tail():
        run(masked=True)

        col = bc_ref[...] + lane
        bv = bv_ref[...]
        m = jnp.max(bv, axis=1, keepdims=True)
        idx = jnp.min(jnp.where(bv == m, col, _BIG_I32), axis=1, keepdims=True)
        out_ref[...] = idx


@functools.partial(jax.jit, static_argnames=("interpret",))
def _sample(logits, interpret=False):
    out = pl.pallas_call(
        _sample_block_kernel,
        grid=(_NBLK,),
        in_specs=[pl.BlockSpec((_B, _C), lambda j: (0, j))],
        out_specs=pl.BlockSpec((_B, 1), lambda j: (0, 0)),
        out_shape=jax.ShapeDtypeStruct((_B, 1), jnp.int32),
        scratch_shapes=[
            pltpu.VMEM((_B, _LANES), jnp.float32),
            pltpu.VMEM((_B, _LANES), jnp.int32),
        ],
        interpret=interpret,
    )(logits)
    return out.reshape(_B).astype(jnp.int64)


def kernel(logits):
    return _sample(logits)


# speedup vs baseline: 1.0680x; 1.0007x over previous
import functools

import jax
import jax.numpy as jnp
import numpy as np
from jax import lax
from jax.experimental import pallas as pl
from jax.experimental.pallas import tpu as pltpu

_B = 64
_N = 1_000_000
_C = 65536
_NBLK = (_N + _C - 1) // _C
_LANES = 128
_CHUNKS = _C // _LANES

_KS0 = np.uint32(0)
_KS1 = np.uint32(42)
_KS2 = np.uint32(0 ^ 42 ^ 0x1BD11BDA)
_ROTS = ((13, 15, 26, 6), (17, 29, 16, 24))

_TINY = np.float32(np.finfo(np.float32).tiny)
_NEG_LN2 = np.float32(-0.6931471805599453)
_NEG_INF = np.float32(-np.inf)
_BIG_I32 = np.int32(2**31 - 1)


def _rotl(v, r):
    return lax.shift_left(v, np.uint32(r)) ^ lax.shift_right_logical(
        v, np.uint32(32 - r)
    )


def _threefry_uniform(x1):
    x0 = x1
    x1 = _rotl(x1, _ROTS[0][0]) ^ x0
    first = True
    ks = (_KS0, _KS1, _KS2)
    for i in range(5):
        for r in _ROTS[i % 2]:
            if first:
                first = False
                continue
            x0 = x0 + x1
            x1 = _rotl(x1, r) ^ x0
        kx = ks[(i + 1) % 3]
        if kx:
            x0 = x0 + kx
        x1 = x1 + np.uint32(ks[(i + 2) % 3] + np.uint32(i + 1))
    bits = x0 ^ x1

    float_bits = lax.shift_right_logical(bits, np.uint32(9)) | np.uint32(0x3F800000)
    floats = lax.bitcast_convert_type(float_bits, jnp.float32) - np.float32(1.0)
    return lax.max(floats, _TINY)


def _sample_block_kernel(logits_ref, out_ref, bv_ref, bc_ref):
    j = pl.program_id(0)

    valid = jnp.minimum(_C, _N - j * _C)

    row = lax.broadcasted_iota(jnp.int32, (_B, _LANES), 0)
    lane = lax.broadcasted_iota(jnp.int32, (_B, _LANES), 1)
    x1_base = (row * _N + lane).astype(jnp.uint32) + (
        (j * _C).astype(jnp.uint32) + _KS1
    )

    @pl.when(j == 0)
    def _init():
        bv_ref[...] = jnp.full((_B, _LANES), _NEG_INF, jnp.float32)
        bc_ref[...] = jnp.zeros((_B, _LANES), jnp.int32)
        out_ref[...] = jnp.zeros((_B, 1), jnp.int32)

    def score(u, off, bv, bc, masked):
        g = -jnp.log(-jnp.log(u))
        s = g + logits_ref[:, pl.ds(off, _LANES)]
        if masked:
            s = jnp.where(lane < valid - off, s, _NEG_INF)
        upd = s > bv
        bv = jnp.maximum(s, bv)
        bc = jnp.where(upd, j * _C + off, bc)
        return bv, bc

    def run(masked):
        def body(k, carry):
            bv, bc, u_prev = carry
            off = k * _LANES
            u = _threefry_uniform(x1_base + off.astype(jnp.uint32))
            bv, bc = score(u_prev, off - _LANES, bv, bc, masked)
            return bv, bc, u

        nch = _CHUNKS if not masked else (valid + _LANES - 1) // _LANES
        u0 = _threefry_uniform(x1_base)
        bv, bc, u_last = lax.fori_loop(
            1,
            nch,
            body,
            (bv_ref[...], bc_ref[...], u0),
            unroll=64 if not masked else None,
        )
        bv, bc = score(u_last, (nch - 1) * _LANES, bv, bc, masked)
        bv_ref[...] = bv
        bc_ref[...] = bc

    @pl.when(j < _NBLK - 1)
    def _full():
        run(masked=False)

    @pl.when(j == _NBLK - 1)
    def _tail():
        run(masked=True)

        col = bc_ref[...] + lane
        bv = bv_ref[...]
        m = jnp.max(bv, axis=1, keepdims=True)
        idx = jnp.min(jnp.where(bv == m, col, _BIG_I32), axis=1, keepdims=True)
        out_ref[...] = idx


@functools.partial(jax.jit, static_argnames=("interpret",))
def _sample(logits, interpret=False):
    out = pl.pallas_call(
        _sample_block_kernel,
        grid=(_NBLK,),
        in_specs=[pl.BlockSpec((_B, _C), lambda j: (0, j))],
        out_specs=pl.BlockSpec((_B, 1), lambda j: (0, 0)),
        out_shape=jax.ShapeDtypeStruct((_B, 1), jnp.int32),
        scratch_shapes=[
            pltpu.VMEM((_B, _LANES), jnp.float32),
            pltpu.VMEM((_B, _LANES), jnp.int32),
        ],
        interpret=interpret,
    )(logits)
    return out.reshape(_B).astype(jnp.int64)


def kernel(logits):
    return _sample(logits)
